# Initial kernel scaffold; baseline (speedup 1.0000x reference)
import jax
import jax.numpy as jnp
from jax.experimental import pallas as pl


def kernel(token_ids, pos_table, nest_table, seg_table, W):
    raise NotImplementedError("write your pallas kernel here")


# speedup vs baseline: 6.0161x; 6.0161x over previous
import functools

import jax
import jax.numpy as jnp
from jax import lax
from jax.experimental import pallas as pl
from jax.experimental.pallas import tpu as pltpu
from jax.experimental.pallas import tpu_sc as plsc

B, S, H = 4, 2048, 512
NLEV, NSEG = 16, 8
NG_ROWS = NLEV * NSEG
LANES = 16


def _shifted(x, k, fill):
    pad = jnp.full((B, k), fill, x.dtype)
    return jnp.concatenate([pad, x[:, :-k]], axis=1)


def _prefix(x, op, fill):
    k = 1
    while k < S:
        x = op(x, _shifted(x, k, fill))
        k *= 2
    return x


def _tables_kernel(tok_ref, pos_ref, nest_ref, seg_ref, wt_ref,
                   p_ref, ng_ref, idx_ref):
    f32 = jnp.float32
    p_ref[...] = jnp.dot(pos_ref[...], wt_ref[0:H, :],
                         preferred_element_type=f32)
    n_proj = jnp.dot(nest_ref[...], wt_ref[H:2 * H, :],
                     preferred_element_type=f32)
    g_proj = jnp.dot(seg_ref[...], wt_ref[2 * H:3 * H, :],
                     preferred_element_type=f32)
    rn = lax.broadcasted_iota(jnp.int32, (NG_ROWS, NLEV), 0)
    cn = lax.broadcasted_iota(jnp.int32, (NG_ROWS, NLEV), 1)
    sel_n = ((rn // NSEG) == cn).astype(f32)
    rg = lax.broadcasted_iota(jnp.int32, (NG_ROWS, NSEG), 0)
    cg = lax.broadcasted_iota(jnp.int32, (NG_ROWS, NSEG), 1)
    sel_g = ((rg % NSEG) == cg).astype(f32)
    ng_ref[...] = (jnp.dot(sel_n, n_proj, preferred_element_type=f32)
                   + jnp.dot(sel_g, g_proj, preferred_element_type=f32))

    tok = tok_ref[...]
    is_open = (tok == 40) | (tok == 123) | (tok == 91)
    is_close = (tok == 41) | (tok == 125) | (tok == 93)
    d = jnp.where(is_open, 1, 0) + jnp.where(is_close, -1, 0)
    s_sum = _prefix(d, jnp.add, 0)
    s_min = _prefix(s_sum, jnp.minimum, 2 ** 30)
    level = s_sum - jnp.minimum(s_min, 0)
    nest_idx = jnp.minimum(level, NLEV - 1)
    trig = jnp.where(tok > 39990, 1, 0)
    seg_idx = jnp.bitwise_and(_prefix(trig, jnp.add, 0), NSEG - 1)
    idx_ref[...] = nest_idx * NSEG + seg_idx


def _tables(tok, pos, nest, seg, wt):
    return pl.pallas_call(
        _tables_kernel,
        out_shape=(
            jax.ShapeDtypeStruct((S, H), jnp.float32),
            jax.ShapeDtypeStruct((NG_ROWS, H), jnp.float32),
            jax.ShapeDtypeStruct((B, S), jnp.int32),
        ),
    )(tok, pos, nest, seg, wt)


def _combine(p, ng, idx):
    info = plsc.get_sparse_core_info()
    nw = info.num_cores * info.num_subcores
    ch = S // nw
    mesh = plsc.VectorSubcoreMesh(core_axis_name="c", subcore_axis_name="s")

    @functools.partial(
        pl.kernel, mesh=mesh,
        out_type=jax.ShapeDtypeStruct((B, S, H), jnp.float32),
        scratch_types=[
            pltpu.VMEM((ch,), jnp.int32),
            pltpu.VMEM((ch, H), jnp.float32),
            pltpu.VMEM((ch, H), jnp.float32),
            pltpu.SemaphoreType.DMA,
        ],
    )
    def scatter_combine(p_hbm, ng_hbm, idx_hbm, out_hbm, idx_v, p_v, g_v, sem):
        wid = lax.axis_index("s") * info.num_cores + lax.axis_index("c")
        s0 = wid * ch
        pltpu.sync_copy(p_hbm.at[pl.ds(s0, ch)], p_v)
        for b in range(B):
            pltpu.sync_copy(idx_hbm.at[b, pl.ds(s0, ch)], idx_v)
            pltpu.async_copy(ng_hbm.at[idx_v], g_v, sem).wait()

            def body(r, carry):
                for c in range(H // LANES):
                    sl = pl.ds(c * LANES, LANES)
                    g_v[r, sl] = g_v[r, sl] + p_v[r, sl]
                return carry

            lax.fori_loop(0, ch, body, 0)
            pltpu.sync_copy(g_v, out_hbm.at[b, pl.ds(s0, ch)])

    return scatter_combine(p, ng, idx)


def kernel(token_ids, pos_table, nest_table, seg_table, W):
    tok = token_ids.astype(jnp.int32)
    p, ng, idx = _tables(tok, pos_table, nest_table, seg_table, W.T)
    return _combine(p, ng, idx)


# speedup vs baseline: 6.6037x; 1.0977x over previous
import functools

import jax
import jax.numpy as jnp
from jax import lax
from jax.experimental import pallas as pl
from jax.experimental.pallas import tpu as pltpu
from jax.experimental.pallas import tpu_sc as plsc

B, S, H = 4, 2048, 512
NLEV, NSEG = 16, 8
NG_ROWS = NLEV * NSEG
LANES = 16


def _shifted(x, k, fill):
    pad = jnp.full((B, k), fill, x.dtype)
    return jnp.concatenate([pad, x[:, :-k]], axis=1)


def _prefix(x, op, fill):
    k = 1
    while k < S:
        x = op(x, _shifted(x, k, fill))
        k *= 2
    return x


def _tables_kernel(tok_ref, pos_ref, nest_ref, seg_ref, wt_ref,
                   p_ref, ng_ref, idx_ref):
    f32 = jnp.float32
    p_ref[...] = jnp.dot(pos_ref[...], wt_ref[0:H, :],
                         preferred_element_type=f32)
    n_proj = jnp.dot(nest_ref[...], wt_ref[H:2 * H, :],
                     preferred_element_type=f32)
    g_proj = jnp.dot(seg_ref[...], wt_ref[2 * H:3 * H, :],
                     preferred_element_type=f32)
    rn = lax.broadcasted_iota(jnp.int32, (NG_ROWS, NLEV), 0)
    cn = lax.broadcasted_iota(jnp.int32, (NG_ROWS, NLEV), 1)
    sel_n = ((rn // NSEG) == cn).astype(f32)
    rg = lax.broadcasted_iota(jnp.int32, (NG_ROWS, NSEG), 0)
    cg = lax.broadcasted_iota(jnp.int32, (NG_ROWS, NSEG), 1)
    sel_g = ((rg % NSEG) == cg).astype(f32)
    ng_ref[...] = (jnp.dot(sel_n, n_proj, preferred_element_type=f32)
                   + jnp.dot(sel_g, g_proj, preferred_element_type=f32))

    tok = tok_ref[...]
    is_open = (tok == 40) | (tok == 123) | (tok == 91)
    is_close = (tok == 41) | (tok == 125) | (tok == 93)
    d = jnp.where(is_open, 1, 0) + jnp.where(is_close, -1, 0)
    s_sum = _prefix(d, jnp.add, 0)
    s_min = _prefix(s_sum, jnp.minimum, 2 ** 30)
    level = s_sum - jnp.minimum(s_min, 0)
    nest_idx = jnp.minimum(level, NLEV - 1)
    trig = jnp.where(tok > 39990, 1, 0)
    seg_idx = jnp.bitwise_and(_prefix(trig, jnp.add, 0), NSEG - 1)
    idx_ref[...] = nest_idx * NSEG + seg_idx


def _tables(tok, pos, nest, seg, wt):
    return pl.pallas_call(
        _tables_kernel,
        out_shape=(
            jax.ShapeDtypeStruct((S, H), jnp.float32),
            jax.ShapeDtypeStruct((NG_ROWS, H), jnp.float32),
            jax.ShapeDtypeStruct((B, S), jnp.int32),
        ),
    )(tok, pos, nest, seg, wt)


NBUF = 4
SUB = 2


def _combine(p, ng, idx):
    info = plsc.get_sparse_core_info()
    nw = info.num_cores * info.num_subcores
    ch = S // nw
    rows = ch // SUB
    nchunks = B * SUB
    mesh = plsc.VectorSubcoreMesh(core_axis_name="c", subcore_axis_name="s")

    @functools.partial(
        pl.kernel, mesh=mesh,
        out_type=jax.ShapeDtypeStruct((B, S, H), jnp.float32),
        scratch_types=[
            pltpu.VMEM((B * ch,), jnp.int32),
            pltpu.VMEM((ch, H), jnp.float32),
            pltpu.VMEM((NBUF, rows, H), jnp.float32),
            pltpu.SemaphoreType.DMA,
            [pltpu.SemaphoreType.DMA] * NBUF,
            [pltpu.SemaphoreType.DMA] * NBUF,
        ],
    )
    def scatter_combine(p_hbm, ng_hbm, idx_hbm, out_hbm,
                        idx_v, p_v, g_v, psem, gsems, wsems):
        wid = lax.axis_index("s") * info.num_cores + lax.axis_index("c")
        s0 = wid * ch
        for b in range(B):
            pltpu.sync_copy(idx_hbm.at[b, pl.ds(s0, ch)],
                            idx_v.at[pl.ds(b * ch, ch)])
        pcopy = pltpu.async_copy(p_hbm.at[pl.ds(s0, ch)], p_v, psem)

        def gather(t):
            b, off = t // SUB, (t % SUB) * rows
            return pltpu.async_copy(
                ng_hbm.at[idx_v.at[pl.ds(b * ch + off, rows)]],
                g_v.at[t % NBUF], gsems[t % NBUF])

        gc = [gather(t) for t in range(NBUF)]
        wc = [None] * NBUF
        pcopy.wait()
        for t in range(nchunks):
            buf = t % NBUF
            b, off = t // SUB, (t % SUB) * rows
            gc[buf].wait()

            def body(r, carry):
                for c in range(H // LANES):
                    sl = pl.ds(c * LANES, LANES)
                    plsc.addupdate(g_v.at[buf, r, sl], p_v[off + r, sl])
                return carry

            lax.fori_loop(0, rows, body, 0, unroll=4)
            wc[buf] = pltpu.async_copy(
                g_v.at[buf], out_hbm.at[b, pl.ds(s0 + off, rows)], wsems[buf])
            if t + NBUF < nchunks:
                wc[buf].wait()
                gc[buf] = gather(t + NBUF)
        for t in range(nchunks - NBUF, nchunks):
            wc[t % NBUF].wait()

    return scatter_combine(p, ng, idx)


def kernel(token_ids, pos_table, nest_table, seg_table, W):
    tok = token_ids.astype(jnp.int32)
    p, ng, idx = _tables(tok, pos_table, nest_table, seg_table, W.T)
    return _combine(p, ng, idx)


# speedup vs baseline: 20.3617x; 3.0834x over previous
import functools

import jax
import jax.numpy as jnp
from jax import lax
from jax.experimental import pallas as pl
from jax.experimental.pallas import tpu as pltpu
from jax.experimental.pallas import tpu_sc as plsc

B, S, H = 4, 2048, 512
NLEV, NSEG = 16, 8
NG_ROWS = NLEV * NSEG
LANES = 16


def _shifted(x, k, fill):
    pad = jnp.full((B, k), fill, x.dtype)
    return jnp.concatenate([pad, x[:, :-k]], axis=1)


def _prefix(x, op, fill):
    k = 1
    while k < S:
        x = op(x, _shifted(x, k, fill))
        k *= 2
    return x


def _tables_kernel(tok_ref, pos_ref, nest_ref, seg_ref, wt_ref,
                   p_ref, ng_ref, idx_ref):
    f32 = jnp.float32
    p_ref[...] = jnp.dot(pos_ref[...], wt_ref[0:H, :],
                         preferred_element_type=f32)
    n_proj = jnp.dot(nest_ref[...], wt_ref[H:2 * H, :],
                     preferred_element_type=f32)
    g_proj = jnp.dot(seg_ref[...], wt_ref[2 * H:3 * H, :],
                     preferred_element_type=f32)
    rn = lax.broadcasted_iota(jnp.int32, (NG_ROWS, NLEV), 0)
    cn = lax.broadcasted_iota(jnp.int32, (NG_ROWS, NLEV), 1)
    sel_n = ((rn // NSEG) == cn).astype(f32)
    rg = lax.broadcasted_iota(jnp.int32, (NG_ROWS, NSEG), 0)
    cg = lax.broadcasted_iota(jnp.int32, (NG_ROWS, NSEG), 1)
    sel_g = ((rg % NSEG) == cg).astype(f32)
    ng_ref[...] = (jnp.dot(sel_n, n_proj, preferred_element_type=f32)
                   + jnp.dot(sel_g, g_proj, preferred_element_type=f32))

    tok = tok_ref[...]
    is_open = (tok == 40) | (tok == 123) | (tok == 91)
    is_close = (tok == 41) | (tok == 125) | (tok == 93)
    d = jnp.where(is_open, 1, 0) + jnp.where(is_close, -1, 0)
    s_sum = _prefix(d, jnp.add, 0)
    s_min = _prefix(s_sum, jnp.minimum, 2 ** 30)
    level = s_sum - jnp.minimum(s_min, 0)
    nest_idx = jnp.minimum(level, NLEV - 1)
    trig = jnp.where(tok > 39990, 1, 0)
    seg_idx = jnp.bitwise_and(_prefix(trig, jnp.add, 0), NSEG - 1)
    idx_ref[...] = nest_idx * NSEG + seg_idx


def _tables(tok, pos, nest, seg, wt):
    return pl.pallas_call(
        _tables_kernel,
        out_shape=(
            jax.ShapeDtypeStruct((S, H), jnp.float32),
            jax.ShapeDtypeStruct((NG_ROWS, H), jnp.float32),
            jax.ShapeDtypeStruct((B, S), jnp.int32),
        ),
    )(tok, pos, nest, seg, wt)


NBUF = 2
CROWS = 16


def _combine(p, ng, idx):
    info = plsc.get_sparse_core_info()
    nw = info.num_cores * info.num_subcores
    ch = S // nw
    nchunks = B * ch // CROWS
    per_b = ch // CROWS
    mesh = plsc.VectorSubcoreMesh(core_axis_name="c", subcore_axis_name="s")

    @functools.partial(
        pl.kernel, mesh=mesh,
        out_type=jax.ShapeDtypeStruct((B, S, H), jnp.float32),
        scratch_types=[
            pltpu.VMEM((B * ch + LANES,), jnp.int32),
            pltpu.VMEM((NG_ROWS, H), jnp.float32),
            pltpu.VMEM((ch, H), jnp.float32),
            pltpu.VMEM((NBUF, CROWS, H), jnp.float32),
            pltpu.SemaphoreType.DMA,
            pltpu.SemaphoreType.DMA,
            [pltpu.SemaphoreType.DMA] * NBUF,
        ],
    )
    def scatter_combine(p_hbm, ng_hbm, idx_hbm, out_hbm,
                        idx_v, ng_v, p_v, o_v, ngsem, psem, wsems):
        wid = lax.axis_index("s") * info.num_cores + lax.axis_index("c")
        s0 = wid * ch
        ngc = pltpu.async_copy(ng_hbm, ng_v, ngsem)
        pc = pltpu.async_copy(p_hbm.at[pl.ds(s0, ch)], p_v, psem)
        for b in range(B):
            pltpu.sync_copy(idx_hbm.at[b, pl.ds(s0, ch)],
                            idx_v.at[pl.ds(b * ch, ch)])
        ngc.wait()
        pc.wait()
        wc = [None] * NBUF
        for t in range(nchunks):
            buf = t % NBUF
            b, off = t // per_b, (t % per_b) * CROWS
            if t >= NBUF:
                wc[buf].wait()

            def body(r, carry):
                k = idx_v[pl.ds(b * ch + off + r, LANES)][0]
                for c in range(H // LANES):
                    sl = pl.ds(c * LANES, LANES)
                    o_v[buf, r, sl] = ng_v[k, sl] + p_v[off + r, sl]
                return carry

            lax.fori_loop(0, CROWS, body, 0, unroll=2)
            wc[buf] = pltpu.async_copy(
                o_v.at[buf], out_hbm.at[b, pl.ds(s0 + off, CROWS)], wsems[buf])
        for t in range(NBUF):
            wc[t].wait()

    return scatter_combine(p, ng, idx)


def kernel(token_ids, pos_table, nest_table, seg_table, W):
    tok = token_ids.astype(jnp.int32)
    p, ng, idx = _tables(tok, pos_table, nest_table, seg_table, W.T)
    return _combine(p, ng, idx)


# speedup vs baseline: 21.0374x; 1.0332x over previous
import functools

import jax
import jax.numpy as jnp
from jax import lax
from jax.experimental import pallas as pl
from jax.experimental.pallas import tpu as pltpu
from jax.experimental.pallas import tpu_sc as plsc

B, S, H = 4, 2048, 512
NLEV, NSEG = 16, 8
NG_ROWS = NLEV * NSEG
LANES = 16


def _shifted(x, k, fill):
    pad = jnp.full((B, k), fill, x.dtype)
    return jnp.concatenate([pad, x[:, :-k]], axis=1)


def _prefix(x, op, fill):
    k = 1
    while k < S:
        x = op(x, _shifted(x, k, fill))
        k *= 2
    return x


def _tables_kernel(tok_ref, pos_ref, nest_ref, seg_ref, wt_ref,
                   p_ref, ng_ref, idx_ref):
    f32 = jnp.float32
    p_ref[...] = jnp.dot(pos_ref[...], wt_ref[0:H, :],
                         preferred_element_type=f32)
    n_proj = jnp.dot(nest_ref[...], wt_ref[H:2 * H, :],
                     preferred_element_type=f32)
    g_proj = jnp.dot(seg_ref[...], wt_ref[2 * H:3 * H, :],
                     preferred_element_type=f32)
    rn = lax.broadcasted_iota(jnp.int32, (NG_ROWS, NLEV), 0)
    cn = lax.broadcasted_iota(jnp.int32, (NG_ROWS, NLEV), 1)
    sel_n = ((rn // NSEG) == cn).astype(f32)
    rg = lax.broadcasted_iota(jnp.int32, (NG_ROWS, NSEG), 0)
    cg = lax.broadcasted_iota(jnp.int32, (NG_ROWS, NSEG), 1)
    sel_g = ((rg % NSEG) == cg).astype(f32)
    ng_ref[...] = (jnp.dot(sel_n, n_proj, preferred_element_type=f32)
                   + jnp.dot(sel_g, g_proj, preferred_element_type=f32))

    tok = tok_ref[...]
    is_open = (tok == 40) | (tok == 123) | (tok == 91)
    is_close = (tok == 41) | (tok == 125) | (tok == 93)
    d = jnp.where(is_open, 1, 0) + jnp.where(is_close, -1, 0)
    s_sum = _prefix(d, jnp.add, 0)
    s_min = _prefix(s_sum, jnp.minimum, 2 ** 30)
    level = s_sum - jnp.minimum(s_min, 0)
    nest_idx = jnp.minimum(level, NLEV - 1)
    trig = jnp.where(tok > 39990, 1, 0)
    seg_idx = jnp.bitwise_and(_prefix(trig, jnp.add, 0), NSEG - 1)
    idx_ref[...] = nest_idx * NSEG + seg_idx


def _tables(tok, pos, nest, seg, wt):
    return pl.pallas_call(
        _tables_kernel,
        out_shape=(
            jax.ShapeDtypeStruct((S, H), jnp.float32),
            jax.ShapeDtypeStruct((NG_ROWS, H), jnp.float32),
            jax.ShapeDtypeStruct((B, S), jnp.int32),
        ),
    )(tok, pos, nest, seg, wt)


NBUF = 2
RCH = 8


def _combine(p, ng, idx):
    info = plsc.get_sparse_core_info()
    nw = info.num_cores * info.num_subcores
    ch = S // nw
    nchunks = ch // RCH
    mesh = plsc.VectorSubcoreMesh(core_axis_name="c", subcore_axis_name="s")

    @functools.partial(
        pl.kernel, mesh=mesh,
        out_type=jax.ShapeDtypeStruct((B, S, H), jnp.float32),
        scratch_types=[
            pltpu.VMEM((B * ch + LANES,), jnp.int32),
            pltpu.VMEM((NG_ROWS, H), jnp.float32),
            pltpu.VMEM((NBUF, RCH, H), jnp.float32),
            pltpu.VMEM((NBUF, B, RCH, H), jnp.float32),
            pltpu.SemaphoreType.DMA,
            pltpu.SemaphoreType.DMA,
            [pltpu.SemaphoreType.DMA] * NBUF,
            [pltpu.SemaphoreType.DMA] * NBUF,
        ],
    )
    def scatter_combine(p_hbm, ng_hbm, idx_hbm, out_hbm,
                        idx_v, ng_v, p_v, o_v, ngsem, isem, psems, wsems):
        wid = lax.axis_index("s") * info.num_cores + lax.axis_index("c")
        s0 = wid * ch
        ngc = pltpu.async_copy(ng_hbm, ng_v, ngsem)

        def pchunk(j):
            return pltpu.async_copy(
                p_hbm.at[pl.ds(s0 + j * RCH, RCH)], p_v.at[j % NBUF],
                psems[j % NBUF])

        pcs = [pchunk(0), pchunk(1)]
        ics = [pltpu.async_copy(idx_hbm.at[b, pl.ds(s0, ch)],
                                idx_v.at[pl.ds(b * ch, ch)], isem)
               for b in range(B)]
        for c in ics:
            c.wait()
        ngc.wait()
        wcs = [None] * NBUF
        for j in range(nchunks):
            slot = j % NBUF
            off = j * RCH
            pcs[slot].wait()
            if j >= NBUF:
                for c in wcs[slot]:
                    c.wait()

            def body(r, carry):
                ks = [idx_v[pl.ds(b * ch + off + r, LANES)][0]
                      for b in range(B)]
                for c in range(H // LANES):
                    sl = pl.ds(c * LANES, LANES)
                    pv = p_v[slot, r, sl]
                    for b in range(B):
                        o_v[slot, b, r, sl] = ng_v[ks[b], sl] + pv
                return carry

            lax.fori_loop(0, RCH, body, 0, unroll=2)
            wcs[slot] = [
                pltpu.async_copy(o_v.at[slot, b],
                                 out_hbm.at[b, pl.ds(s0 + off, RCH)],
                                 wsems[slot])
                for b in range(B)]
            if j + NBUF < nchunks:
                pcs[slot] = pchunk(j + NBUF)
        for slot in range(NBUF):
            for c in wcs[slot]:
                c.wait()

    return scatter_combine(p, ng, idx)


def kernel(token_ids, pos_table, nest_table, seg_table, W):
    tok = token_ids.astype(jnp.int32)
    p, ng, idx = _tables(tok, pos_table, nest_table, seg_table, W.T)
    return _combine(p, ng, idx)


# speedup vs baseline: 25.0743x; 1.1919x over previous
import functools

import jax
import jax.numpy as jnp
from jax import lax
from jax.experimental import pallas as pl
from jax.experimental.pallas import tpu as pltpu
from jax.experimental.pallas import tpu_sc as plsc

B, S, H = 4, 2048, 512
NLEV, NSEG = 16, 8
NG_ROWS = NLEV * NSEG
LANES = 16


def _shifted(x, k, fill):
    pad = jnp.full((B, k), fill, x.dtype)
    return jnp.concatenate([pad, x[:, :-k]], axis=1)


def _prefix(x, op, fill):
    k = 1
    while k < S:
        x = op(x, _shifted(x, k, fill))
        k *= 2
    return x


def _tables_kernel(tok_ref, pos_ref, nest_ref, seg_ref, wt_ref,
                   p_ref, ng_ref, idx_ref):
    f32 = jnp.float32
    p_ref[...] = jnp.dot(pos_ref[...], wt_ref[0:H, :],
                         preferred_element_type=f32)
    n_proj = jnp.dot(nest_ref[...], wt_ref[H:2 * H, :],
                     preferred_element_type=f32)
    g_proj = jnp.dot(seg_ref[...], wt_ref[2 * H:3 * H, :],
                     preferred_element_type=f32)
    rn = lax.broadcasted_iota(jnp.int32, (NG_ROWS, NLEV), 0)
    cn = lax.broadcasted_iota(jnp.int32, (NG_ROWS, NLEV), 1)
    sel_n = ((rn // NSEG) == cn).astype(f32)
    rg = lax.broadcasted_iota(jnp.int32, (NG_ROWS, NSEG), 0)
    cg = lax.broadcasted_iota(jnp.int32, (NG_ROWS, NSEG), 1)
    sel_g = ((rg % NSEG) == cg).astype(f32)
    ng_ref[...] = (jnp.dot(sel_n, n_proj, preferred_element_type=f32)
                   + jnp.dot(sel_g, g_proj, preferred_element_type=f32))

    tok = tok_ref[...]
    is_open = (tok == 40) | (tok == 123) | (tok == 91)
    is_close = (tok == 41) | (tok == 125) | (tok == 93)
    d = jnp.where(is_open, 1, 0) + jnp.where(is_close, -1, 0)
    s_sum = _prefix(d, jnp.add, 0)
    s_min = _prefix(s_sum, jnp.minimum, 2 ** 30)
    level = s_sum - jnp.minimum(s_min, 0)
    nest_idx = jnp.minimum(level, NLEV - 1)
    trig = jnp.where(tok > 39990, 1, 0)
    seg_idx = jnp.bitwise_and(_prefix(trig, jnp.add, 0), NSEG - 1)
    idx_ref[...] = nest_idx * NSEG + seg_idx


def _tables(tok, pos, nest, seg, wt):
    return pl.pallas_call(
        _tables_kernel,
        out_shape=(
            jax.ShapeDtypeStruct((S, H), jnp.float32),
            jax.ShapeDtypeStruct((NG_ROWS, H), jnp.float32),
            jax.ShapeDtypeStruct((B, S), jnp.int32),
        ),
    )(tok, pos, nest, seg, wt)


NBUF = 2
RCH = 8


def _combine(p, ng, idx):
    info = plsc.get_sparse_core_info()
    nw = info.num_cores * info.num_subcores
    ch = S // nw
    nchunks = ch // RCH
    mesh = plsc.VectorSubcoreMesh(core_axis_name="c", subcore_axis_name="s")

    @functools.partial(
        pl.kernel, mesh=mesh,
        out_type=jax.ShapeDtypeStruct((B, S, H), jnp.float32),
        scratch_types=[
            pltpu.VMEM((B * ch + LANES,), jnp.int32),
            pltpu.VMEM((NG_ROWS, H), jnp.float32),
            pltpu.VMEM((NBUF, RCH, H), jnp.float32),
            pltpu.VMEM((NBUF, B, RCH, H), jnp.float32),
            pltpu.SemaphoreType.DMA,
            pltpu.SemaphoreType.DMA,
            [pltpu.SemaphoreType.DMA] * NBUF,
            [pltpu.SemaphoreType.DMA] * NBUF,
        ],
    )
    def scatter_combine(p_hbm, ng_hbm, idx_hbm, out_hbm,
                        idx_v, ng_v, p_v, o_v, ngsem, isem, psems, wsems):
        wid = lax.axis_index("s") * info.num_cores + lax.axis_index("c")
        s0 = wid * ch
        ngc = pltpu.async_copy(ng_hbm, ng_v, ngsem)

        def pchunk(j):
            return pltpu.async_copy(
                p_hbm.at[pl.ds(s0 + j * RCH, RCH)], p_v.at[j % NBUF],
                psems[j % NBUF])

        pcs = [pchunk(0), pchunk(1)]
        ics = [pltpu.async_copy(idx_hbm.at[b, pl.ds(s0, ch)],
                                idx_v.at[pl.ds(b * ch, ch)], isem)
               for b in range(B)]
        for c in ics:
            c.wait()
        ngc.wait()
        wcs = [None] * NBUF
        for j in range(nchunks):
            slot = j % NBUF
            off = j * RCH
            pcs[slot].wait()
            if j >= NBUF:
                for c in wcs[slot]:
                    c.wait()

            @plsc.parallel_loop(0, RCH, step=1, unroll=2)
            def body(r):
                ks = [idx_v[pl.ds(b * ch + off + r, LANES)][0]
                      for b in range(B)]
                for c in range(H // LANES):
                    sl = pl.ds(c * LANES, LANES)
                    pv = p_v[slot, r, sl]
                    for b in range(B):
                        o_v[slot, b, r, sl] = ng_v[ks[b], sl] + pv
            wcs[slot] = [
                pltpu.async_copy(o_v.at[slot, b],
                                 out_hbm.at[b, pl.ds(s0 + off, RCH)],
                                 wsems[slot])
                for b in range(B)]
            if j + NBUF < nchunks:
                pcs[slot] = pchunk(j + NBUF)
        for slot in range(NBUF):
            for c in wcs[slot]:
                c.wait()

    return scatter_combine(p, ng, idx)


def kernel(token_ids, pos_table, nest_table, seg_table, W):
    tok = token_ids.astype(jnp.int32)
    p, ng, idx = _tables(tok, pos_table, nest_table, seg_table, W.T)
    return _combine(p, ng, idx)


# speedup vs baseline: 28.7648x; 1.1472x over previous
import functools

import jax
import jax.numpy as jnp
from jax import lax
from jax.experimental import pallas as pl
from jax.experimental.pallas import tpu as pltpu
from jax.experimental.pallas import tpu_sc as plsc

B, S, H = 4, 2048, 512
NLEV, NSEG = 16, 8
NG_ROWS = NLEV * NSEG
LANES = 16


def _shifted(x, k, fill):
    pad = jnp.full((B, k), fill, x.dtype)
    return jnp.concatenate([pad, x[:, :-k]], axis=1)


def _prefix(x, op, fill):
    k = 1
    while k < S:
        x = op(x, _shifted(x, k, fill))
        k *= 2
    return x


def _tables_kernel(tok_ref, pos_ref, nest_ref, seg_ref, wt_ref,
                   p_ref, ng_ref, idx_ref):
    f32 = jnp.float32
    p_ref[...] = jnp.dot(pos_ref[...], wt_ref[0:H, :],
                         preferred_element_type=f32)
    n_proj = jnp.dot(nest_ref[...], wt_ref[H:2 * H, :],
                     preferred_element_type=f32)
    g_proj = jnp.dot(seg_ref[...], wt_ref[2 * H:3 * H, :],
                     preferred_element_type=f32)
    rn = lax.broadcasted_iota(jnp.int32, (NG_ROWS, NLEV), 0)
    cn = lax.broadcasted_iota(jnp.int32, (NG_ROWS, NLEV), 1)
    sel_n = ((rn // NSEG) == cn).astype(f32)
    rg = lax.broadcasted_iota(jnp.int32, (NG_ROWS, NSEG), 0)
    cg = lax.broadcasted_iota(jnp.int32, (NG_ROWS, NSEG), 1)
    sel_g = ((rg % NSEG) == cg).astype(f32)
    ng_ref[...] = (jnp.dot(sel_n, n_proj, preferred_element_type=f32)
                   + jnp.dot(sel_g, g_proj, preferred_element_type=f32))

    tok = tok_ref[...]
    is_open = (tok == 40) | (tok == 123) | (tok == 91)
    is_close = (tok == 41) | (tok == 125) | (tok == 93)
    d = jnp.where(is_open, 1, 0) + jnp.where(is_close, -1, 0)
    s_sum = _prefix(d, jnp.add, 0)
    s_min = _prefix(s_sum, jnp.minimum, 2 ** 30)
    level = s_sum - jnp.minimum(s_min, 0)
    nest_idx = jnp.minimum(level, NLEV - 1)
    trig = jnp.where(tok > 39990, 1, 0)
    seg_idx = jnp.bitwise_and(_prefix(trig, jnp.add, 0), NSEG - 1)
    idx_ref[...] = nest_idx * NSEG + seg_idx


def _tables(tok, pos, nest, seg, wt):
    return pl.pallas_call(
        _tables_kernel,
        out_shape=(
            jax.ShapeDtypeStruct((S, H), jnp.float32),
            jax.ShapeDtypeStruct((NG_ROWS, H), jnp.float32),
            jax.ShapeDtypeStruct((B, S), jnp.int32),
        ),
    )(tok, pos, nest, seg, wt)


NBUF = 3
RCH = 8


def _combine(p, ng, idx):
    info = plsc.get_sparse_core_info()
    nw = info.num_cores * info.num_subcores
    ch = S // nw
    nchunks = ch // RCH
    rpc = B * RCH
    mesh = plsc.VectorSubcoreMesh(core_axis_name="c", subcore_axis_name="s")

    @functools.partial(
        pl.kernel, mesh=mesh,
        out_type=jax.ShapeDtypeStruct((B, S, H), jnp.float32),
        scratch_types=[
            pltpu.VMEM((B * ch + LANES,), jnp.int32),
            pltpu.VMEM((NG_ROWS, H), jnp.float32),
            pltpu.VMEM((NBUF, B, RCH, H), jnp.float32),
            pltpu.SemaphoreType.DMA,
            pltpu.SemaphoreType.DMA,
            [pltpu.SemaphoreType.DMA] * NBUF,
            [pltpu.SemaphoreType.DMA] * NBUF,
        ],
    )
    def scatter_combine(p_hbm, ng_hbm, idx_hbm, out_hbm,
                        idx_v, ng_v, o_v, ngsem, isem, psems, wsems):
        wid = lax.axis_index("s") * info.num_cores + lax.axis_index("c")
        s0 = wid * ch
        ngc = pltpu.async_copy(ng_hbm, ng_v, ngsem)

        def fill(j):
            return [pltpu.async_copy(p_hbm.at[pl.ds(s0 + j * RCH, RCH)],
                                     o_v.at[j % NBUF, b], psems[j % NBUF])
                    for b in range(B)]

        fills = [fill(0), fill(1), None]
        ics = [pltpu.async_copy(idx_hbm.at[b, pl.ds(s0, ch)],
                                idx_v.at[pl.ds(b * ch, ch)], isem)
               for b in range(B)]
        for c in ics:
            c.wait()
        ngc.wait()
        wcs = [None] * NBUF
        for j in range(nchunks):
            slot = j % NBUF
            off = j * RCH
            if j + 2 < nchunks:
                pre = (j + 2) % NBUF
                if wcs[pre] is not None:
                    for c in wcs[pre]:
                        c.wait()
                fills[pre] = fill(j + 2)
            for c in fills[slot]:
                c.wait()

            @plsc.parallel_loop(0, rpc, step=1, unroll=4)
            def body(i):
                b = i >> 3
                r = i & (RCH - 1)
                k = idx_v[pl.ds(b * ch + off + r, LANES)][0]
                for c in range(H // LANES):
                    sl = pl.ds(c * LANES, LANES)
                    plsc.addupdate(o_v.at[slot, b, r, sl], ng_v[k, sl])
            wcs[slot] = [
                pltpu.async_copy(o_v.at[slot, b],
                                 out_hbm.at[b, pl.ds(s0 + off, RCH)],
                                 wsems[slot])
                for b in range(B)]
        for slot in range(NBUF):
            if wcs[slot] is not None:
                for c in wcs[slot]:
                    c.wait()

    return scatter_combine(p, ng, idx)


def kernel(token_ids, pos_table, nest_table, seg_table, W):
    tok = token_ids.astype(jnp.int32)
    p, ng, idx = _tables(tok, pos_table, nest_table, seg_table, W.T)
    return _combine(p, ng, idx)
